# Initial kernel scaffold; baseline (speedup 1.0000x reference)
import jax
import jax.numpy as jnp
from jax.experimental import pallas as pl


def kernel(color_tbl, shape_tbl, color_idx, shape_idx):
    raise NotImplementedError("write your pallas kernel here")


# speedup vs baseline: 2.0916x; 2.0916x over previous
import jax
import jax.numpy as jnp
from jax import lax
from jax.experimental import pallas as pl
from jax.experimental.pallas import tpu as pltpu
from jax.experimental.pallas import tpu_sc as plsc

NUM_COLORS = 100
NUM_SHAPES = 100
D = 128
BATCH = 16384

NC = 2
NS = 16
NW = NC * NS
BPW = BATCH // NW
CHUNK = 128
NCH = BPW // CHUNK


def _body(color_tbl, shape_tbl, cidx, sidx, out, idx_v, rows_v, gsem, wsem):
    wid = lax.axis_index("s") * NC + lax.axis_index("c")
    base = wid * BPW

    pltpu.sync_copy(cidx.at[wid], idx_v.at[0])
    pltpu.sync_copy(sidx.at[wid], idx_v.at[1])

    for t, tbl in ((0, color_tbl), (1, shape_tbl)):
        for j in range(NCH):
            pltpu.async_copy(tbl.at[idx_v.at[t, j]], rows_v, gsem).wait()
            pltpu.sync_copy(rows_v, out.at[t, pl.ds(base + j * CHUNK, CHUNK)])


def kernel(color_tbl, shape_tbl, color_idx, shape_idx):
    cidx = color_idx.reshape(NW, NCH, CHUNK)
    sidx = shape_idx.reshape(NW, NCH, CHUNK)
    mesh = plsc.VectorSubcoreMesh(core_axis_name="c", subcore_axis_name="s")
    f = pl.kernel(
        _body,
        out_type=jax.ShapeDtypeStruct((2, BATCH, D), jnp.float32),
        mesh=mesh,
        scratch_types=[
            pltpu.VMEM((2, NCH, CHUNK), jnp.int32),
            pltpu.VMEM((CHUNK, D), jnp.float32),
            pltpu.SemaphoreType.DMA,
            pltpu.SemaphoreType.DMA,
        ],
    )
    return f(color_tbl, shape_tbl, cidx, sidx)


# speedup vs baseline: 2.1953x; 1.0496x over previous
import jax
import jax.numpy as jnp
from jax import lax
from jax.experimental import pallas as pl
from jax.experimental.pallas import tpu as pltpu
from jax.experimental.pallas import tpu_sc as plsc

NUM_COLORS = 100
NUM_SHAPES = 100
D = 128
BATCH = 16384

NC = 2
NS = 16
NW = NC * NS
BPW = BATCH // NW
CHUNK = 128
NCH = BPW // CHUNK


NBUF = 4


def _body(color_tbl, shape_tbl, cidx, sidx, out, idx_v, rows_v, gsem, wsem):
    wid = lax.axis_index("s") * NC + lax.axis_index("c")
    base = wid * BPW

    pltpu.sync_copy(cidx.at[wid], idx_v.at[0])
    pltpu.sync_copy(sidx.at[wid], idx_v.at[1])

    chunks = [(t, tbl, j) for t, tbl in ((0, color_tbl), (1, shape_tbl))
              for j in range(NCH)]
    nck = len(chunks)
    g = [None] * nck
    w = [None] * nck

    def fire_write(k):
        t, _, j = chunks[k]
        return pltpu.async_copy(
            rows_v.at[k % NBUF],
            out.at[t, pl.ds(base + j * CHUNK, CHUNK)],
            wsem.at[k % NBUF])

    for k in range(nck):
        p = k % NBUF
        if k >= NBUF:
            w[k - NBUF].wait()
        t, tbl, j = chunks[k]
        g[k] = pltpu.async_copy(tbl.at[idx_v.at[t, j]], rows_v.at[p],
                                gsem.at[p])
        if k >= 1:
            g[k - 1].wait()
            w[k - 1] = fire_write(k - 1)
    g[nck - 1].wait()
    w[nck - 1] = fire_write(nck - 1)
    for k in range(nck - NBUF, nck):
        w[k].wait()


def kernel(color_tbl, shape_tbl, color_idx, shape_idx):
    cidx = color_idx.reshape(NW, NCH, CHUNK)
    sidx = shape_idx.reshape(NW, NCH, CHUNK)
    mesh = plsc.VectorSubcoreMesh(core_axis_name="c", subcore_axis_name="s")
    f = pl.kernel(
        _body,
        out_type=jax.ShapeDtypeStruct((2, BATCH, D), jnp.float32),
        mesh=mesh,
        scratch_types=[
            pltpu.VMEM((2, NCH, CHUNK), jnp.int32),
            pltpu.VMEM((NBUF, CHUNK, D), jnp.float32),
            pltpu.SemaphoreType.DMA((NBUF,)),
            pltpu.SemaphoreType.DMA((NBUF,)),
        ],
    )
    return f(color_tbl, shape_tbl, cidx, sidx)


# speedup vs baseline: 4.3444x; 1.9790x over previous
import jax
import jax.numpy as jnp
from jax import lax
from jax.experimental import pallas as pl
from jax.experimental.pallas import tpu as pltpu
from jax.experimental.pallas import tpu_sc as plsc

NUM_COLORS = 100
NUM_SHAPES = 100
D = 128
BATCH = 16384

NC = 2
NS = 16
NW = NC * NS
BPW = BATCH // NW
CHUNK = 128
NCH = BPW // CHUNK


NBUF = 4


def _body(color_tbl, shape_tbl, cidx, sidx, out, idx_v, ctbl_v, stbl_v,
          rows_v, gsem, wsem):
    wid = lax.axis_index("s") * NC + lax.axis_index("c")
    base = wid * BPW

    pltpu.sync_copy(cidx.at[wid], idx_v.at[0])
    pltpu.sync_copy(sidx.at[wid], idx_v.at[1])
    @pl.when(lax.axis_index("s") == 0)
    def _stage():
        pltpu.sync_copy(color_tbl, ctbl_v)
        pltpu.sync_copy(shape_tbl, stbl_v)
    plsc.subcore_barrier()

    chunks = [(t, tbl, j) for t, tbl in ((0, ctbl_v), (1, stbl_v))
              for j in range(NCH)]
    nck = len(chunks)
    g = [None] * nck
    w = [None] * nck

    def fire_write(k):
        t, _, j = chunks[k]
        return pltpu.async_copy(
            rows_v.at[k % NBUF],
            out.at[t, pl.ds(base + j * CHUNK, CHUNK)],
            wsem.at[k % NBUF])

    for k in range(nck):
        p = k % NBUF
        if k >= NBUF:
            w[k - NBUF].wait()
        t, tbl, j = chunks[k]
        g[k] = pltpu.async_copy(tbl.at[idx_v.at[t, j]], rows_v.at[p],
                                gsem.at[p])
        if k >= 1:
            g[k - 1].wait()
            w[k - 1] = fire_write(k - 1)
    g[nck - 1].wait()
    w[nck - 1] = fire_write(nck - 1)
    for k in range(nck - NBUF, nck):
        w[k].wait()


def kernel(color_tbl, shape_tbl, color_idx, shape_idx):
    cidx = color_idx.reshape(NW, NCH, CHUNK)
    sidx = shape_idx.reshape(NW, NCH, CHUNK)
    mesh = plsc.VectorSubcoreMesh(core_axis_name="c", subcore_axis_name="s")
    f = pl.kernel(
        _body,
        out_type=jax.ShapeDtypeStruct((2, BATCH, D), jnp.float32),
        mesh=mesh,
        scratch_types=[
            pltpu.VMEM((2, NCH, CHUNK), jnp.int32),
            pltpu.VMEM_SHARED((NUM_COLORS, D), jnp.float32),
            pltpu.VMEM_SHARED((NUM_SHAPES, D), jnp.float32),
            pltpu.VMEM((NBUF, CHUNK, D), jnp.float32),
            pltpu.SemaphoreType.DMA((NBUF,)),
            pltpu.SemaphoreType.DMA((NBUF,)),
        ],
    )
    return f(color_tbl, shape_tbl, cidx, sidx)
